# Initial kernel scaffold; baseline (speedup 1.0000x reference)
import jax
import jax.numpy as jnp
from jax.experimental import pallas as pl


def kernel(input_ids, table, W, b):
    raise NotImplementedError("write your pallas kernel here")


# speedup vs baseline: 1.1424x; 1.1424x over previous
import functools

import jax
import jax.numpy as jnp
from jax import lax
from jax.experimental import pallas as pl
from jax.experimental.pallas import tpu as pltpu
from jax.experimental.pallas import tpu_sc as plsc

VOCAB = 100000
HIDDEN = 1024
NUM_TOKEN_CUTOFF = 1000
LIN_ROWS = 1024
L = 16
NC, NS = 2, 16
NW = NC * NS
TOKENS = 4 * 2048
BPW = TOKENS // NW
CH = 64


def _linear_body(x_ref, w_ref, b_ref, o_ref):
    o_ref[...] = lax.dot_general(
        x_ref[...], w_ref[...],
        dimension_numbers=(((1,), (1,)), ((), ())),
        preferred_element_type=jnp.float32,
    ) + b_ref[...]


def _lin_table(table, w, b2d):
    tile = 256
    return pl.pallas_call(
        _linear_body,
        grid=(LIN_ROWS // tile,),
        in_specs=[
            pl.BlockSpec((tile, HIDDEN), lambda i: (i, 0)),
            pl.BlockSpec((HIDDEN, HIDDEN), lambda i: (0, 0)),
            pl.BlockSpec((1, HIDDEN), lambda i: (0, 0)),
        ],
        out_specs=pl.BlockSpec((tile, HIDDEN), lambda i: (i, 0)),
        out_shape=jax.ShapeDtypeStruct((LIN_ROWS, HIDDEN), jnp.float32),
    )(table, w, b2d)


def _sc_body(ids_hbm, table_hbm, lin_hbm, out_hbm, ids_v, rows_v, lrows_v,
             gsem, ssem):
    wid = lax.axis_index("s") * NC + lax.axis_index("c")
    base = wid * BPW
    pltpu.sync_copy(ids_hbm.at[pl.ds(base, BPW)], ids_v)

    for c in range(BPW // CH):
        pltpu.async_copy(
            table_hbm.at[ids_v.at[pl.ds(c * CH, CH)]], rows_v, gsem).wait()
        pltpu.sync_copy(rows_v, out_hbm.at[pl.ds(base + c * CH, CH)])

    lane = lax.iota(jnp.int32, L)
    for v in range(BPW // L):
        idv = ids_v[pl.ds(v * L, L)]
        mask = idv < NUM_TOKEN_CUTOFF
        ffs = plsc.all_reduce_ffs(mask)

        @pl.when(ffs[0] < L)
        def _(idv=idv, mask=mask, ffs=ffs, v=v):
            fill_id = plsc.load_gather(ids_v, [v * L + ffs])
            gidx = jnp.where(mask, idv, fill_id)
            posv = base + v * L + jnp.where(mask, lane, ffs)
            pltpu.async_copy(lin_hbm.at[gidx], lrows_v, gsem).wait()
            pltpu.async_copy(lrows_v, out_hbm.at[posv], ssem).wait()


_sc_embed = functools.partial(
    pl.kernel,
    out_type=jax.ShapeDtypeStruct((TOKENS, HIDDEN), jnp.float32),
    mesh=plsc.VectorSubcoreMesh(core_axis_name="c", subcore_axis_name="s"),
    compiler_params=pltpu.CompilerParams(needs_layout_passes=False),
    scratch_types=[
        pltpu.VMEM((BPW,), jnp.int32),
        pltpu.VMEM((CH, HIDDEN), jnp.float32),
        pltpu.VMEM((L, HIDDEN), jnp.float32),
        pltpu.SemaphoreType.DMA,
        pltpu.SemaphoreType.DMA,
    ],
)(_sc_body)


@jax.jit
def kernel(input_ids, table, W, b):
    ids_flat = input_ids.reshape(-1).astype(jnp.int32)
    lin = _lin_table(table, W, b.reshape(1, HIDDEN))
    out = _sc_embed(ids_flat, table, lin)
    return out.reshape(input_ids.shape + (HIDDEN,))


# speedup vs baseline: 1.1641x; 1.0190x over previous
import functools

import jax
import jax.numpy as jnp
from jax import lax
from jax.experimental import pallas as pl
from jax.experimental.pallas import tpu as pltpu
from jax.experimental.pallas import tpu_sc as plsc

VOCAB = 100000
HIDDEN = 1024
NUM_TOKEN_CUTOFF = 1000
LIN_ROWS = 1024
L = 16
NC, NS = 2, 16
NW = NC * NS
TOKENS = 4 * 2048
BPW = TOKENS // NW
CH = 32


def _linear_body(x_ref, w_ref, b_ref, o_ref):
    o_ref[...] = lax.dot_general(
        x_ref[...], w_ref[...],
        dimension_numbers=(((1,), (1,)), ((), ())),
        preferred_element_type=jnp.float32,
    ) + b_ref[...]


def _lin_table(table, w, b2d):
    tile = 256
    return pl.pallas_call(
        _linear_body,
        grid=(LIN_ROWS // tile,),
        in_specs=[
            pl.BlockSpec((tile, HIDDEN), lambda i: (i, 0)),
            pl.BlockSpec((HIDDEN, HIDDEN), lambda i: (0, 0)),
            pl.BlockSpec((1, HIDDEN), lambda i: (0, 0)),
        ],
        out_specs=pl.BlockSpec((tile, HIDDEN), lambda i: (i, 0)),
        out_shape=jax.ShapeDtypeStruct((LIN_ROWS, HIDDEN), jnp.float32),
    )(table, w, b2d)


def _sc_body(ids_hbm, table_hbm, lin_hbm, out_hbm, ids_v, rows0_v, rows1_v,
             lrows_v, gsem0, gsem1, ssem0, ssem1):
    wid = lax.axis_index("s") * NC + lax.axis_index("c")
    base = wid * BPW
    pltpu.sync_copy(ids_hbm.at[pl.ds(base, BPW)], ids_v)

    rows = (rows0_v, rows1_v)
    gsem = (gsem0, gsem1)
    ssem = (ssem0, ssem1)
    nchunk = BPW // CH

    def gather(c):
        return pltpu.async_copy(
            table_hbm.at[ids_v.at[pl.ds(c * CH, CH)]], rows[c & 1], gsem[c & 1])

    def store(c):
        return pltpu.async_copy(
            rows[c & 1], out_hbm.at[pl.ds(base + c * CH, CH)], ssem[c & 1])

    stores = [None] * nchunk
    gathers = [None] * nchunk
    gathers[0] = gather(0)
    for c in range(nchunk):
        if c + 1 < nchunk:
            if c >= 1:
                stores[c - 1].wait()
            gathers[c + 1] = gather(c + 1)
        gathers[c].wait()
        stores[c] = store(c)
    stores[nchunk - 2].wait()
    stores[nchunk - 1].wait()

    lane = lax.iota(jnp.int32, L)
    for v in range(BPW // L):
        idv = ids_v[pl.ds(v * L, L)]
        mask = idv < NUM_TOKEN_CUTOFF
        ffs = plsc.all_reduce_ffs(mask)

        @pl.when(ffs[0] < L)
        def _(idv=idv, mask=mask, ffs=ffs, v=v):
            fill_id = plsc.load_gather(ids_v, [v * L + ffs])
            gidx = jnp.where(mask, idv, fill_id)
            posv = base + v * L + jnp.where(mask, lane, ffs)
            pltpu.async_copy(lin_hbm.at[gidx], lrows_v, gsem0).wait()
            pltpu.async_copy(lrows_v, out_hbm.at[posv], ssem0).wait()


_sc_embed = functools.partial(
    pl.kernel,
    out_type=jax.ShapeDtypeStruct((TOKENS, HIDDEN), jnp.float32),
    mesh=plsc.VectorSubcoreMesh(core_axis_name="c", subcore_axis_name="s"),
    compiler_params=pltpu.CompilerParams(needs_layout_passes=False),
    scratch_types=[
        pltpu.VMEM((BPW,), jnp.int32),
        pltpu.VMEM((CH, HIDDEN), jnp.float32),
        pltpu.VMEM((CH, HIDDEN), jnp.float32),
        pltpu.VMEM((L, HIDDEN), jnp.float32),
        pltpu.SemaphoreType.DMA,
        pltpu.SemaphoreType.DMA,
        pltpu.SemaphoreType.DMA,
        pltpu.SemaphoreType.DMA,
    ],
)(_sc_body)


@jax.jit
def kernel(input_ids, table, W, b):
    ids_flat = input_ids.reshape(-1).astype(jnp.int32)
    lin = _lin_table(table, W, b.reshape(1, HIDDEN))
    out = _sc_embed(ids_flat, table, lin)
    return out.reshape(input_ids.shape + (HIDDEN,))


# speedup vs baseline: 1.1720x; 1.0068x over previous
import functools

import jax
import jax.numpy as jnp
from jax import lax
from jax.experimental import pallas as pl
from jax.experimental.pallas import tpu as pltpu
from jax.experimental.pallas import tpu_sc as plsc

VOCAB = 100000
HIDDEN = 1024
NUM_TOKEN_CUTOFF = 1000
LIN_ROWS = 1024
L = 16
NC, NS = 2, 16
NW = NC * NS
TOKENS = 4 * 2048
BPW = TOKENS // NW
CH = 32
NBUF = 3


def _linear_body(x_ref, w_ref, b_ref, o_ref):
    o_ref[...] = lax.dot_general(
        x_ref[...], w_ref[...],
        dimension_numbers=(((1,), (1,)), ((), ())),
        preferred_element_type=jnp.float32,
    ) + b_ref[...]


def _lin_table(table, w, b2d):
    tile = 256
    return pl.pallas_call(
        _linear_body,
        grid=(LIN_ROWS // tile,),
        in_specs=[
            pl.BlockSpec((tile, HIDDEN), lambda i: (i, 0)),
            pl.BlockSpec((HIDDEN, HIDDEN), lambda i: (0, 0)),
            pl.BlockSpec((1, HIDDEN), lambda i: (0, 0)),
        ],
        out_specs=pl.BlockSpec((tile, HIDDEN), lambda i: (i, 0)),
        out_shape=jax.ShapeDtypeStruct((LIN_ROWS, HIDDEN), jnp.float32),
    )(table, w, b2d)


def _sc_body(ids_hbm, table_hbm, lin_hbm, out_hbm, ids_v, rows_v,
             lrows_v, gsem0, gsem1, gsem2, ssem0, ssem1, ssem2):
    wid = lax.axis_index("s") * NC + lax.axis_index("c")
    base = wid * BPW
    rows = tuple(rows_v.at[b] for b in range(NBUF))
    gsem = (gsem0, gsem1, gsem2)
    ssem = (ssem0, ssem1, ssem2)
    pltpu.sync_copy(ids_hbm.at[pl.ds(base, BPW)], ids_v)

    nchunk = BPW // CH

    def gather(c):
        b = c % NBUF
        return pltpu.async_copy(
            table_hbm.at[ids_v.at[pl.ds(c * CH, CH)]], rows[b], gsem[b])

    def store(c):
        b = c % NBUF
        return pltpu.async_copy(
            rows[b], out_hbm.at[pl.ds(base + c * CH, CH)], ssem[b])

    stores = [None] * nchunk
    gathers = [None] * nchunk
    for c in range(min(NBUF, nchunk)):
        gathers[c] = gather(c)
    for c in range(nchunk):
        gathers[c].wait()
        stores[c] = store(c)
        if c + NBUF < nchunk:
            stores[c].wait()
            gathers[c + NBUF] = gather(c + NBUF)
    for c in range(max(0, nchunk - NBUF), nchunk):
        stores[c].wait()

    lane = lax.iota(jnp.int32, L)
    for v in range(BPW // L):
        idv = ids_v[pl.ds(v * L, L)]
        mask = idv < NUM_TOKEN_CUTOFF
        ffs = plsc.all_reduce_ffs(mask)

        @pl.when(ffs[0] < L)
        def _(idv=idv, mask=mask, ffs=ffs, v=v):
            fill_id = plsc.load_gather(ids_v, [v * L + ffs])
            gidx = jnp.where(mask, idv, fill_id)
            posv = base + v * L + jnp.where(mask, lane, ffs)
            pltpu.async_copy(lin_hbm.at[gidx], lrows_v, gsem0).wait()
            pltpu.async_copy(lrows_v, out_hbm.at[posv], ssem0).wait()


_sc_embed = functools.partial(
    pl.kernel,
    out_type=jax.ShapeDtypeStruct((TOKENS, HIDDEN), jnp.float32),
    mesh=plsc.VectorSubcoreMesh(core_axis_name="c", subcore_axis_name="s"),
    compiler_params=pltpu.CompilerParams(needs_layout_passes=False),
    scratch_types=[
        pltpu.VMEM((BPW,), jnp.int32),
        pltpu.VMEM((NBUF, CH, HIDDEN), jnp.float32),
        pltpu.VMEM((L, HIDDEN), jnp.float32),
        pltpu.SemaphoreType.DMA,
        pltpu.SemaphoreType.DMA,
        pltpu.SemaphoreType.DMA,
        pltpu.SemaphoreType.DMA,
        pltpu.SemaphoreType.DMA,
        pltpu.SemaphoreType.DMA,
    ],
)(_sc_body)


@jax.jit
def kernel(input_ids, table, W, b):
    ids_flat = input_ids.reshape(-1).astype(jnp.int32)
    lin = _lin_table(table, W, b.reshape(1, HIDDEN))
    out = _sc_embed(ids_flat, table, lin)
    return out.reshape(input_ids.shape + (HIDDEN,))


# speedup vs baseline: 1.4361x; 1.2253x over previous
import functools

import jax
import jax.numpy as jnp
from jax import lax
from jax.experimental import pallas as pl
from jax.experimental.pallas import tpu as pltpu
from jax.experimental.pallas import tpu_sc as plsc

VOCAB = 100000
HIDDEN = 1024
NUM_TOKEN_CUTOFF = 1000
LIN_ROWS = 1024
L = 16
NC, NS = 2, 16
NW = NC * NS
TOKENS = 4 * 2048
BPW = TOKENS // NW
CH = 32
NBUF = 3
GBUF_N = BPW + 2 * L


def _linear_body(x_ref, w_ref, b_ref, o_ref):
    o_ref[...] = lax.dot_general(
        x_ref[...], w_ref[...],
        dimension_numbers=(((1,), (1,)), ((), ())),
        preferred_element_type=jnp.float32,
    ) + b_ref[...]


def _lin_table(table, w, b2d):
    tile = 256
    return pl.pallas_call(
        _linear_body,
        grid=(LIN_ROWS // tile,),
        in_specs=[
            pl.BlockSpec((tile, HIDDEN), lambda i: (i, 0)),
            pl.BlockSpec((HIDDEN, HIDDEN), lambda i: (0, 0)),
            pl.BlockSpec((1, HIDDEN), lambda i: (0, 0)),
        ],
        out_specs=pl.BlockSpec((tile, HIDDEN), lambda i: (i, 0)),
        out_shape=jax.ShapeDtypeStruct((LIN_ROWS, HIDDEN), jnp.float32),
    )(table, w, b2d)


def _sc_body(ids_hbm, table_hbm, lin_hbm, out_hbm, ids_v, rows_v,
             lrows_v, gbuf_v, pbuf_v, gsem0, gsem1, gsem2,
             ssem0, ssem1, ssem2):
    wid = lax.axis_index("s") * NC + lax.axis_index("c")
    base = wid * BPW
    rows = tuple(rows_v.at[b] for b in range(NBUF))
    gsem = (gsem0, gsem1, gsem2)
    ssem = (ssem0, ssem1, ssem2)
    pltpu.sync_copy(ids_hbm.at[pl.ds(base, BPW)], ids_v)

    nchunk = BPW // CH

    def gather(c):
        b = c % NBUF
        return pltpu.async_copy(
            table_hbm.at[ids_v.at[pl.ds(c * CH, CH)]], rows[b], gsem[b])

    def store(c):
        b = c % NBUF
        return pltpu.async_copy(
            rows[b], out_hbm.at[pl.ds(base + c * CH, CH)], ssem[b])

    stores = [None] * nchunk
    gathers = [None] * nchunk
    for c in range(min(NBUF, nchunk)):
        gathers[c] = gather(c)
    for c in range(nchunk):
        gathers[c].wait()
        stores[c] = store(c)
        if c + NBUF < nchunk:
            stores[c].wait()
            gathers[c + NBUF] = gather(c + NBUF)
    for c in range(max(0, nchunk - NBUF), nchunk):
        stores[c].wait()

    lane = lax.iota(jnp.int32, L)
    for i in range(GBUF_N // L):
        gbuf_v[pl.ds(i * L, L)] = lane * 0
        pbuf_v[pl.ds(i * L, L)] = lane * 0 + base
    cnt = jnp.int32(0)
    for v in range(BPW // L):
        idv = ids_v[pl.ds(v * L, L)]
        mask = idv < NUM_TOKEN_CUTOFF
        cs = plsc.cumsum(mask.astype(jnp.int32))
        dest = cnt + cs - 1
        dest = jnp.where(mask, jnp.maximum(dest, 0), BPW + L + lane)
        plsc.store_scatter(gbuf_v, [dest], idv, mask=mask)
        plsc.store_scatter(pbuf_v, [dest], base + v * L + lane, mask=mask)
        cnt = cnt + plsc.all_reduce_population_count(mask)[0]

    for k in range(BPW // L):
        @pl.when(cnt > k * L)
        def _(k=k):
            gi = gbuf_v[pl.ds(k * L, L)]
            po = pbuf_v[pl.ds(k * L, L)]
            valid = (k * L + lane) < cnt
            gi = jnp.where(valid, gi, gi[0])
            po = jnp.where(valid, po, po[0])
            pltpu.async_copy(lin_hbm.at[gi], lrows_v, gsem0).wait()
            pltpu.async_copy(lrows_v, out_hbm.at[po], ssem0).wait()


_sc_embed = functools.partial(
    pl.kernel,
    out_type=jax.ShapeDtypeStruct((TOKENS, HIDDEN), jnp.float32),
    mesh=plsc.VectorSubcoreMesh(core_axis_name="c", subcore_axis_name="s"),
    compiler_params=pltpu.CompilerParams(needs_layout_passes=False),
    scratch_types=[
        pltpu.VMEM((BPW,), jnp.int32),
        pltpu.VMEM((NBUF, CH, HIDDEN), jnp.float32),
        pltpu.VMEM((L, HIDDEN), jnp.float32),
        pltpu.VMEM((GBUF_N,), jnp.int32),
        pltpu.VMEM((GBUF_N,), jnp.int32),
        pltpu.SemaphoreType.DMA,
        pltpu.SemaphoreType.DMA,
        pltpu.SemaphoreType.DMA,
        pltpu.SemaphoreType.DMA,
        pltpu.SemaphoreType.DMA,
        pltpu.SemaphoreType.DMA,
    ],
)(_sc_body)


@jax.jit
def kernel(input_ids, table, W, b):
    ids_flat = input_ids.reshape(-1).astype(jnp.int32)
    lin = _lin_table(table, W, b.reshape(1, HIDDEN))
    out = _sc_embed(ids_flat, table, lin)
    return out.reshape(input_ids.shape + (HIDDEN,))


# speedup vs baseline: 1.5172x; 1.0565x over previous
import functools

import jax
import jax.numpy as jnp
from jax import lax
from jax.experimental import pallas as pl
from jax.experimental.pallas import tpu as pltpu
from jax.experimental.pallas import tpu_sc as plsc

VOCAB = 100000
HIDDEN = 1024
NUM_TOKEN_CUTOFF = 1000
LIN_ROWS = 1024
L = 16
NC, NS = 2, 16
NW = NC * NS
TOKENS = 4 * 2048
BPW = TOKENS // NW
CH = 32
NBUF = 3
GBUF_N = BPW + 2 * L


def _linear_body(x_ref, w_ref, b_ref, o_ref):
    o_ref[...] = lax.dot_general(
        x_ref[...], w_ref[...],
        dimension_numbers=(((1,), (1,)), ((), ())),
        preferred_element_type=jnp.float32,
    ) + b_ref[...]


def _lin_table(table, w, b2d):
    tile = 256
    return pl.pallas_call(
        _linear_body,
        grid=(LIN_ROWS // tile,),
        in_specs=[
            pl.BlockSpec((tile, HIDDEN), lambda i: (i, 0)),
            pl.BlockSpec((HIDDEN, HIDDEN), lambda i: (0, 0)),
            pl.BlockSpec((1, HIDDEN), lambda i: (0, 0)),
        ],
        out_specs=pl.BlockSpec((tile, HIDDEN), lambda i: (i, 0)),
        out_shape=jax.ShapeDtypeStruct((LIN_ROWS, HIDDEN), jnp.float32),
    )(table, w, b2d)


def _sc_body(ids_hbm, table_hbm, lin_hbm, out_hbm, ids_v, rows_v,
             lrows_v, gbuf_v, pbuf_v, gsem0, gsem1, gsem2,
             ssem0, ssem1, ssem2, mgsem):
    wid = lax.axis_index("s") * NC + lax.axis_index("c")
    base = wid * BPW
    rows = tuple(rows_v.at[b] for b in range(NBUF))
    gsem = (gsem0, gsem1, gsem2)
    ssem = (ssem0, ssem1, ssem2)
    pltpu.sync_copy(ids_hbm.at[pl.ds(base, BPW)], ids_v)

    nchunk = BPW // CH

    def gather(c):
        b = c % NBUF
        return pltpu.async_copy(
            table_hbm.at[ids_v.at[pl.ds(c * CH, CH)]], rows[b], gsem[b])

    def store(c):
        b = c % NBUF
        return pltpu.async_copy(
            rows[b], out_hbm.at[pl.ds(base + c * CH, CH)], ssem[b])

    stores = [None] * nchunk
    gathers = [None] * nchunk
    for c in range(min(NBUF, nchunk)):
        gathers[c] = gather(c)

    lane = lax.iota(jnp.int32, L)
    for i in range(GBUF_N // L):
        gbuf_v[pl.ds(i * L, L)] = lane * 0
        pbuf_v[pl.ds(i * L, L)] = lane * 0 + base
    cnt = jnp.int32(0)
    for v in range(BPW // L):
        idv = ids_v[pl.ds(v * L, L)]
        mask = idv < NUM_TOKEN_CUTOFF
        cs = plsc.cumsum(mask.astype(jnp.int32))
        dest = cnt + cs - 1
        dest = jnp.where(mask, jnp.maximum(dest, 0), BPW + L + lane)
        plsc.store_scatter(gbuf_v, [dest], idv, mask=mask)
        plsc.store_scatter(pbuf_v, [dest], base + v * L + lane, mask=mask)
        cnt = cnt + plsc.all_reduce_population_count(mask)[0]

    def masked_chunk_idx(k):
        gi = gbuf_v[pl.ds(k * L, L)]
        po = pbuf_v[pl.ds(k * L, L)]
        valid = (k * L + lane) < cnt
        return jnp.where(valid, gi, gi[0]), jnp.where(valid, po, po[0])

    @pl.when(cnt > 0)
    def _():
        gi, _po = masked_chunk_idx(0)
        pltpu.async_copy(lin_hbm.at[gi], lrows_v, mgsem)

    for c in range(nchunk):
        gathers[c].wait()
        stores[c] = store(c)
        if c + NBUF < nchunk:
            stores[c].wait()
            gathers[c + NBUF] = gather(c + NBUF)
    for c in range(max(0, nchunk - NBUF), nchunk):
        stores[c].wait()

    @pl.when(cnt > 0)
    def _():
        gi, po = masked_chunk_idx(0)
        pltpu.make_async_copy(lin_hbm.at[gi], lrows_v, mgsem).wait()
        pltpu.async_copy(lrows_v, out_hbm.at[po], ssem0).wait()

    for k in range(1, BPW // L):
        @pl.when(cnt > k * L)
        def _(k=k):
            gi, po = masked_chunk_idx(k)
            pltpu.async_copy(lin_hbm.at[gi], lrows_v, mgsem).wait()
            pltpu.async_copy(lrows_v, out_hbm.at[po], ssem0).wait()


_sc_embed = functools.partial(
    pl.kernel,
    out_type=jax.ShapeDtypeStruct((TOKENS, HIDDEN), jnp.float32),
    mesh=plsc.VectorSubcoreMesh(core_axis_name="c", subcore_axis_name="s"),
    compiler_params=pltpu.CompilerParams(needs_layout_passes=False),
    scratch_types=[
        pltpu.VMEM((BPW,), jnp.int32),
        pltpu.VMEM((NBUF, CH, HIDDEN), jnp.float32),
        pltpu.VMEM((L, HIDDEN), jnp.float32),
        pltpu.VMEM((GBUF_N,), jnp.int32),
        pltpu.VMEM((GBUF_N,), jnp.int32),
        pltpu.SemaphoreType.DMA,
        pltpu.SemaphoreType.DMA,
        pltpu.SemaphoreType.DMA,
        pltpu.SemaphoreType.DMA,
        pltpu.SemaphoreType.DMA,
        pltpu.SemaphoreType.DMA,
        pltpu.SemaphoreType.DMA,
    ],
)(_sc_body)


@jax.jit
def kernel(input_ids, table, W, b):
    ids_flat = input_ids.reshape(-1).astype(jnp.int32)
    lin = _lin_table(table, W, b.reshape(1, HIDDEN))
    out = _sc_embed(ids_flat, table, lin)
    return out.reshape(input_ids.shape + (HIDDEN,))
